# baseline (speedup 1.0000x reference)
import jax
import jax.numpy as jnp
from jax import lax
from jax.experimental import pallas as pl
from jax.experimental.pallas import tpu as pltpu
from jax.experimental.pallas import tpu_sc as plsc

D = 64
B = 16384
H = 20
NC, NS, L = 2, 16, 16
NW = NC * NS
B_PER_W = B // NW
CHUNK = 32
N_CHUNK = B_PER_W // CHUNK
IDX_PER_GATHER = 128
G_PER_CHUNK = CHUNK * H // IDX_PER_GATHER
IDX_ROWS = B_PER_W * H // IDX_PER_GATHER


def _pool_body(pidx_hbm, roff_hbm, tbl_hbm, out_hbm,
               idx_v, roff_v, buf, acc, gsem):
    wid = lax.axis_index("s") * NC + lax.axis_index("c")
    item_base = wid * B_PER_W

    pltpu.sync_copy(pidx_hbm.at[wid], idx_v)
    pltpu.sync_copy(roff_hbm.at[wid], roff_v)

    def chunk_body(c, _):
        descs = [
            pltpu.async_copy(
                tbl_hbm.at[idx_v.at[c * G_PER_CHUNK + g]],
                buf.at[pl.ds(g * IDX_PER_GATHER, IDX_PER_GATHER)],
                gsem)
            for g in range(G_PER_CHUNK)
        ]
        for d in descs:
            d.wait()

        def item_body(i, _):
            def j_body(j, accs):
                r = i * H + j
                rsplat = jnp.full((L,), r, jnp.int32)
                csplat = jnp.full((L,), c, jnp.int32)
                off = plsc.load_gather(roff_v, [csplat, rsplat])
                out = []
                for k in range(D // L):
                    col = off + (k * L + lax.iota(jnp.int32, L))
                    out.append(accs[k] + plsc.load_gather(buf, [rsplat, col]))
                return tuple(out)

            z = jnp.zeros((L,), jnp.float32)
            accs = lax.fori_loop(0, H, j_body, (z,) * (D // L))
            for k in range(D // L):
                acc[i, pl.ds(k * L, L)] = accs[k]
            return 0

        lax.fori_loop(0, CHUNK, item_body, 0)
        pltpu.sync_copy(acc, out_hbm.at[pl.ds(item_base + c * CHUNK, CHUNK)])
        return 0

    lax.fori_loop(0, N_CHUNK, chunk_body, 0)


def _sc_pool(pidx, roff, tbl2):
    mesh = plsc.VectorSubcoreMesh(core_axis_name="c", subcore_axis_name="s")
    return pl.kernel(
        _pool_body,
        out_type=jax.ShapeDtypeStruct((B, 2 * D), jnp.float32),
        mesh=mesh,
        scratch_types=[
            pltpu.VMEM((IDX_ROWS, IDX_PER_GATHER), jnp.int32),
            pltpu.VMEM((N_CHUNK, CHUNK * H), jnp.int32),
            pltpu.VMEM((CHUNK * H, 2 * D), jnp.float32),
            pltpu.VMEM((CHUNK, 2 * D), jnp.float32),
            pltpu.SemaphoreType.DMA,
        ],
        compiler_params=pltpu.CompilerParams(use_tc_tiling_on_sc=True,
                                             needs_layout_passes=False),
        name="cbow_sc_pool",
    )(pidx, roff, tbl2)


def _head_body(pooled_ref, state_ref, stW_ref, stb_ref,
               W1_ref, b1_ref, W2_ref, b2_ref, out_ref):
    x = pooled_ref[:, :D] * (1.0 / H)
    x += lax.dot_general(state_ref[...], stW_ref[...],
                         (((1,), (1,)), ((), ())),
                         preferred_element_type=jnp.float32)
    x += stb_ref[...]
    h = jnp.maximum(x, 0.0)
    h = lax.dot_general(h, W1_ref[...], (((1,), (1,)), ((), ())),
                        preferred_element_type=jnp.float32) + b1_ref[...]
    h = jnp.maximum(h, 0.0)
    out_ref[...] = lax.dot_general(h, W2_ref[...], (((1,), (1,)), ((), ())),
                                   preferred_element_type=jnp.float32) + b2_ref[...]


def _tc_head(pooled, state, state_W, state_b, W1, b1, W2, b2):
    blk = 2048
    grid = (B // blk,)
    full = lambda shape: pl.BlockSpec(shape, lambda i: (0,) * len(shape))
    return pl.pallas_call(
        _head_body,
        grid=grid,
        in_specs=[
            pl.BlockSpec((blk, 2 * D), lambda i: (i, 0)),
            pl.BlockSpec((blk, 3), lambda i: (i, 0)),
            full((D, 3)),
            full((1, D)),
            full((D // 2, D)),
            full((1, D // 2)),
            full((3, D // 2)),
            full((1, 3)),
        ],
        out_specs=pl.BlockSpec((blk, 3), lambda i: (i, 0)),
        out_shape=jax.ShapeDtypeStruct((B, 3), jnp.float32),
        name="cbow_tc_head",
    )(pooled, state, state_W, state_b.reshape(1, D), W1,
      b1.reshape(1, D // 2), W2, b2.reshape(1, 3))


def kernel(players, state, emb_table, state_W, state_b, W1, b1, W2, b2):
    pflat = players.astype(jnp.int32).reshape(-1)
    pidx = (pflat >> 1).reshape(NW, IDX_ROWS, IDX_PER_GATHER)
    roff = ((pflat & 1) << 6).reshape(NW, N_CHUNK, CHUNK * H)
    tbl2 = emb_table.reshape(500000, 2 * D)
    pooled = _sc_pool(pidx, roff, tbl2)
    return _tc_head(pooled, state, state_W, state_b, W1, b1, W2, b2)


# speedup vs baseline: 1.1999x; 1.1999x over previous
import jax
import jax.numpy as jnp
from jax import lax
from jax.experimental import pallas as pl
from jax.experimental.pallas import tpu as pltpu
from jax.experimental.pallas import tpu_sc as plsc

D = 64
B = 16384
H = 20
NC, NS, L = 2, 16, 16
NW = NC * NS
B_PER_W = B // NW
CHUNK = 32
N_CHUNK = B_PER_W // CHUNK
IDX_PER_GATHER = 128
G_PER_CHUNK = CHUNK * H // IDX_PER_GATHER
IDX_ROWS = B_PER_W * H // IDX_PER_GATHER
NUM_ROWS = 1000000
TBLK = 1024
NTBLK = -(-NUM_ROWS // (2 * TBLK))


def _pool_body(pidx_hbm, roff_hbm, tbl_hbm, out_hbm,
               idx_v, roff_v, buf, acc, gsem):
    wid = lax.axis_index("s") * NC + lax.axis_index("c")
    item_base = wid * B_PER_W

    pltpu.sync_copy(pidx_hbm.at[wid], idx_v)
    pltpu.sync_copy(roff_hbm.at[wid], roff_v)

    def chunk_body(c, _):
        descs = [
            pltpu.async_copy(
                tbl_hbm.at[idx_v.at[c * G_PER_CHUNK + g]],
                buf.at[pl.ds(g * IDX_PER_GATHER, IDX_PER_GATHER)],
                gsem)
            for g in range(G_PER_CHUNK)
        ]
        for d in descs:
            d.wait()

        def item_body(i, _):
            def j_body(j, accs):
                r = i * H + j
                rsplat = jnp.full((L,), r, jnp.int32)
                csplat = jnp.full((L,), c, jnp.int32)
                off = plsc.load_gather(roff_v, [csplat, rsplat])
                out = []
                for k in range(D // L):
                    col = off + (k * L + lax.iota(jnp.int32, L))
                    out.append(accs[k] + plsc.load_gather(buf, [rsplat, col]))
                return tuple(out)

            z = jnp.zeros((L,), jnp.float32)
            accs = lax.fori_loop(0, H, j_body, (z,) * (D // L))
            for k in range(D // L):
                acc[i, pl.ds(k * L, L)] = accs[k]
            return 0

        lax.fori_loop(0, CHUNK, item_body, 0)
        pltpu.sync_copy(acc, out_hbm.at[pl.ds(item_base + c * CHUNK, CHUNK)])
        return 0

    lax.fori_loop(0, N_CHUNK, chunk_body, 0)


def _sc_pool(pidx, roff, tbl2):
    mesh = plsc.VectorSubcoreMesh(core_axis_name="c", subcore_axis_name="s")
    return pl.kernel(
        _pool_body,
        out_type=jax.ShapeDtypeStruct((B, 2 * D), jnp.float32),
        mesh=mesh,
        scratch_types=[
            pltpu.VMEM((IDX_ROWS, IDX_PER_GATHER), jnp.int32),
            pltpu.VMEM((N_CHUNK, CHUNK * H), jnp.int32),
            pltpu.VMEM((CHUNK * H, 2 * D), jnp.float32),
            pltpu.VMEM((CHUNK, 2 * D), jnp.float32),
            pltpu.SemaphoreType.DMA,
        ],
        compiler_params=pltpu.CompilerParams(use_tc_tiling_on_sc=True,
                                             needs_layout_passes=False),
        name="cbow_sc_pool",
    )(pidx, roff, tbl2)


def _transpose_body(in_ref, out_ref):
    blk = in_ref[...]
    out_ref[...] = jnp.concatenate(
        [blk[:, :TBLK].T, blk[:, TBLK:].T], axis=1)


def _tc_pair_transpose(emb_t):
    return pl.pallas_call(
        _transpose_body,
        grid=(NTBLK,),
        in_specs=[pl.BlockSpec((D, 2 * TBLK), lambda i: (0, i))],
        out_specs=pl.BlockSpec((TBLK, 2 * D), lambda i: (i, 0)),
        out_shape=jax.ShapeDtypeStruct((NTBLK * TBLK, 2 * D), jnp.float32),
        name="cbow_tc_pairT",
    )(emb_t)


def _head_body(pooled_ref, state_ref, stW_ref, stb_ref,
               W1_ref, b1_ref, W2_ref, b2_ref, out_ref):
    x = pooled_ref[:, :D] * (1.0 / H)
    x += lax.dot_general(state_ref[...], stW_ref[...],
                         (((1,), (1,)), ((), ())),
                         preferred_element_type=jnp.float32)
    x += stb_ref[...]
    h = jnp.maximum(x, 0.0)
    h = lax.dot_general(h, W1_ref[...], (((1,), (1,)), ((), ())),
                        preferred_element_type=jnp.float32) + b1_ref[...]
    h = jnp.maximum(h, 0.0)
    out_ref[...] = lax.dot_general(h, W2_ref[...], (((1,), (1,)), ((), ())),
                                   preferred_element_type=jnp.float32) + b2_ref[...]


def _tc_head(pooled, state, state_W, state_b, W1, b1, W2, b2):
    blk = 2048
    grid = (B // blk,)
    full = lambda shape: pl.BlockSpec(shape, lambda i: (0,) * len(shape))
    return pl.pallas_call(
        _head_body,
        grid=grid,
        in_specs=[
            pl.BlockSpec((blk, 2 * D), lambda i: (i, 0)),
            pl.BlockSpec((blk, 3), lambda i: (i, 0)),
            full((D, 3)),
            full((1, D)),
            full((D // 2, D)),
            full((1, D // 2)),
            full((3, D // 2)),
            full((1, 3)),
        ],
        out_specs=pl.BlockSpec((blk, 3), lambda i: (i, 0)),
        out_shape=jax.ShapeDtypeStruct((B, 3), jnp.float32),
        name="cbow_tc_head",
    )(pooled, state, state_W, state_b.reshape(1, D), W1,
      b1.reshape(1, D // 2), W2, b2.reshape(1, 3))


def kernel(players, state, emb_table, state_W, state_b, W1, b1, W2, b2):
    pflat = players.astype(jnp.int32).reshape(-1)
    pidx = ((pflat // (2 * TBLK)) * TBLK
            + pflat % TBLK).reshape(NW, IDX_ROWS, IDX_PER_GATHER)
    roff = (((pflat // TBLK) % 2) * D).reshape(NW, N_CHUNK, CHUNK * H)
    tbl2 = _tc_pair_transpose(emb_table.T)
    pooled = _sc_pool(pidx, roff, tbl2)
    return _tc_head(pooled, state, state_W, state_b, W1, b1, W2, b2)


# speedup vs baseline: 1.3301x; 1.1085x over previous
import jax
import jax.numpy as jnp
from jax import lax
from jax.experimental import pallas as pl
from jax.experimental.pallas import tpu as pltpu
from jax.experimental.pallas import tpu_sc as plsc

D = 64
B = 16384
H = 20
NC, NS, L = 2, 16, 16
NW = NC * NS
B_PER_W = B // NW
CHUNK = 16
N_CHUNK = B_PER_W // CHUNK
IDX_PER_GATHER = 64
G_PER_CHUNK = CHUNK * H // IDX_PER_GATHER
IDX_ROWS = B_PER_W * H // IDX_PER_GATHER
NUM_ROWS = 1000000
TBLK = 1024
NTBLK = -(-NUM_ROWS // (2 * TBLK))


def _pool_body(pidx_hbm, roff_hbm, tbl_hbm, out_hbm,
               idx_v, roff_v, buf, acc, gsem):
    wid = lax.axis_index("s") * NC + lax.axis_index("c")
    item_base = wid * B_PER_W

    pltpu.sync_copy(pidx_hbm.at[wid], idx_v)
    pltpu.sync_copy(roff_hbm.at[wid], roff_v)

    def fire(c, slot):
        for g in range(G_PER_CHUNK):
            pltpu.async_copy(
                tbl_hbm.at[idx_v.at[c * G_PER_CHUNK + g]],
                buf.at[slot, pl.ds(g * IDX_PER_GATHER, IDX_PER_GATHER)],
                gsem)

    def drain(slot):
        for g in range(G_PER_CHUNK):
            pltpu.make_async_copy(
                tbl_hbm.at[idx_v.at[g]],
                buf.at[slot, pl.ds(g * IDX_PER_GATHER, IDX_PER_GATHER)],
                gsem).wait()

    fire(0, 0)

    def chunk_body(c, _):
        slot = lax.rem(c, 2)

        @pl.when(c + 1 < N_CHUNK)
        def _():
            fire(c + 1, 1 - slot)

        drain(slot)
        ssplat = jnp.full((L,), slot, jnp.int32)
        csplat = jnp.full((L,), c, jnp.int32)

        def item_body(i, _):
            accs = [jnp.zeros((L,), jnp.float32) for _ in range(D // L)]
            for j in range(H):
                r = i * H + j
                rsplat = jnp.full((L,), r, jnp.int32)
                off = plsc.load_gather(roff_v, [csplat, rsplat])
                for k in range(D // L):
                    col = off + (k * L + lax.iota(jnp.int32, L))
                    accs[k] += plsc.load_gather(buf, [ssplat, rsplat, col])
            for k in range(D // L):
                acc[i, pl.ds(k * L, L)] = accs[k]
            return 0

        lax.fori_loop(0, CHUNK, item_body, 0)
        pltpu.sync_copy(acc, out_hbm.at[pl.ds(item_base + c * CHUNK, CHUNK)])
        return 0

    lax.fori_loop(0, N_CHUNK, chunk_body, 0)


def _sc_pool(pidx, roff, tbl2):
    mesh = plsc.VectorSubcoreMesh(core_axis_name="c", subcore_axis_name="s")
    return pl.kernel(
        _pool_body,
        out_type=jax.ShapeDtypeStruct((B, 2 * D), jnp.float32),
        mesh=mesh,
        scratch_types=[
            pltpu.VMEM((IDX_ROWS, IDX_PER_GATHER), jnp.int32),
            pltpu.VMEM((N_CHUNK, CHUNK * H), jnp.int32),
            pltpu.VMEM((2, CHUNK * H, 2 * D), jnp.float32),
            pltpu.VMEM((CHUNK, 2 * D), jnp.float32),
            pltpu.SemaphoreType.DMA,
        ],
        compiler_params=pltpu.CompilerParams(use_tc_tiling_on_sc=True,
                                             needs_layout_passes=False),
        name="cbow_sc_pool",
    )(pidx, roff, tbl2)


def _transpose_body(in_ref, out_ref):
    blk = in_ref[...]
    out_ref[...] = jnp.concatenate(
        [blk[:, :TBLK].T, blk[:, TBLK:].T], axis=1)


def _tc_pair_transpose(emb_t):
    return pl.pallas_call(
        _transpose_body,
        grid=(NTBLK,),
        in_specs=[pl.BlockSpec((D, 2 * TBLK), lambda i: (0, i))],
        out_specs=pl.BlockSpec((TBLK, 2 * D), lambda i: (i, 0)),
        out_shape=jax.ShapeDtypeStruct((NTBLK * TBLK, 2 * D), jnp.float32),
        name="cbow_tc_pairT",
    )(emb_t)


def _head_body(pooled_ref, state_ref, stW_ref, stb_ref,
               W1_ref, b1_ref, W2_ref, b2_ref, out_ref):
    x = pooled_ref[:, :D] * (1.0 / H)
    x += lax.dot_general(state_ref[...], stW_ref[...],
                         (((1,), (1,)), ((), ())),
                         preferred_element_type=jnp.float32)
    x += stb_ref[...]
    h = jnp.maximum(x, 0.0)
    h = lax.dot_general(h, W1_ref[...], (((1,), (1,)), ((), ())),
                        preferred_element_type=jnp.float32) + b1_ref[...]
    h = jnp.maximum(h, 0.0)
    out_ref[...] = lax.dot_general(h, W2_ref[...], (((1,), (1,)), ((), ())),
                                   preferred_element_type=jnp.float32) + b2_ref[...]


def _tc_head(pooled, state, state_W, state_b, W1, b1, W2, b2):
    blk = 2048
    grid = (B // blk,)
    full = lambda shape: pl.BlockSpec(shape, lambda i: (0,) * len(shape))
    return pl.pallas_call(
        _head_body,
        grid=grid,
        in_specs=[
            pl.BlockSpec((blk, 2 * D), lambda i: (i, 0)),
            pl.BlockSpec((blk, 3), lambda i: (i, 0)),
            full((D, 3)),
            full((1, D)),
            full((D // 2, D)),
            full((1, D // 2)),
            full((3, D // 2)),
            full((1, 3)),
        ],
        out_specs=pl.BlockSpec((blk, 3), lambda i: (i, 0)),
        out_shape=jax.ShapeDtypeStruct((B, 3), jnp.float32),
        name="cbow_tc_head",
    )(pooled, state, state_W, state_b.reshape(1, D), W1,
      b1.reshape(1, D // 2), W2, b2.reshape(1, 3))


def kernel(players, state, emb_table, state_W, state_b, W1, b1, W2, b2):
    pflat = players.astype(jnp.int32).reshape(-1)
    pidx = ((pflat // (2 * TBLK)) * TBLK
            + pflat % TBLK).reshape(NW, IDX_ROWS, IDX_PER_GATHER)
    roff = (((pflat // TBLK) % 2) * D).reshape(NW, N_CHUNK, CHUNK * H)
    tbl2 = _tc_pair_transpose(emb_table.T)
    pooled = _sc_pool(pidx, roff, tbl2)
    return _tc_head(pooled, state, state_W, state_b, W1, b1, W2, b2)


# speedup vs baseline: 1.6712x; 1.2564x over previous
import jax
import jax.numpy as jnp
from jax import lax
from jax.experimental import pallas as pl
from jax.experimental.pallas import tpu as pltpu
from jax.experimental.pallas import tpu_sc as plsc

D = 64
B = 16384
H = 20
NC, NS, L = 2, 16, 16
NW = NC * NS
B_PER_W = B // NW
CHUNK = 16
N_CHUNK = B_PER_W // CHUNK
IDX_PER_GATHER = 64
G_PER_CHUNK = CHUNK * H // IDX_PER_GATHER
IDX_ROWS = B_PER_W * H // IDX_PER_GATHER
NUM_ROWS = 1000000
TBLK = 2048
NTBLK = -(-NUM_ROWS // (2 * TBLK))


def _pool_body(pidx_hbm, roff_hbm, tbl_hbm, out_hbm,
               idx_v, roff_v, buf, acc, gsem):
    wid = lax.axis_index("s") * NC + lax.axis_index("c")
    item_base = wid * B_PER_W

    pltpu.sync_copy(pidx_hbm.at[wid], idx_v)
    pltpu.sync_copy(roff_hbm.at[wid], roff_v)

    def fire(c, slot):
        for g in range(G_PER_CHUNK):
            pltpu.async_copy(
                tbl_hbm.at[idx_v.at[c * G_PER_CHUNK + g]],
                buf.at[slot, pl.ds(g * IDX_PER_GATHER, IDX_PER_GATHER)],
                gsem)

    def drain(slot):
        for g in range(G_PER_CHUNK):
            pltpu.make_async_copy(
                tbl_hbm.at[idx_v.at[g]],
                buf.at[slot, pl.ds(g * IDX_PER_GATHER, IDX_PER_GATHER)],
                gsem).wait()

    fire(0, 0)

    def chunk_body(c, _):
        slot = lax.rem(c, 2)

        @pl.when(c + 1 < N_CHUNK)
        def _():
            fire(c + 1, 1 - slot)

        drain(slot)
        ssplat = jnp.full((L,), slot, jnp.int32)
        csplat = jnp.full((L,), c, jnp.int32)

        def item_body(i, _):
            accs = [jnp.zeros((L,), jnp.float32) for _ in range(D // L)]
            for j in range(H):
                r = i * H + j
                rsplat = jnp.full((L,), r, jnp.int32)
                off = plsc.load_gather(roff_v, [csplat, rsplat])
                for k in range(D // L):
                    col = off + (k * L + lax.iota(jnp.int32, L))
                    accs[k] += plsc.load_gather(buf, [ssplat, rsplat, col])
            for k in range(D // L):
                acc[i, pl.ds(k * L, L)] = accs[k]
            return 0

        lax.fori_loop(0, CHUNK, item_body, 0)
        pltpu.sync_copy(acc, out_hbm.at[pl.ds(item_base + c * CHUNK, CHUNK)])
        return 0

    lax.fori_loop(0, N_CHUNK, chunk_body, 0)


def _sc_pool(pidx, roff, tbl2):
    mesh = plsc.VectorSubcoreMesh(core_axis_name="c", subcore_axis_name="s")
    return pl.kernel(
        _pool_body,
        out_type=jax.ShapeDtypeStruct((B, 2 * D), jnp.float32),
        mesh=mesh,
        scratch_types=[
            pltpu.VMEM((IDX_ROWS, IDX_PER_GATHER), jnp.int32),
            pltpu.VMEM((N_CHUNK, CHUNK * H), jnp.int32),
            pltpu.VMEM((2, CHUNK * H, 2 * D), jnp.float32),
            pltpu.VMEM((CHUNK, 2 * D), jnp.float32),
            pltpu.SemaphoreType.DMA,
        ],
        compiler_params=pltpu.CompilerParams(use_tc_tiling_on_sc=True,
                                             needs_layout_passes=False),
        name="cbow_sc_pool",
    )(pidx, roff, tbl2)


def _transpose_body(in_ref, out_ref):
    blk = in_ref[...]
    out_ref[...] = jnp.concatenate(
        [blk[:, :TBLK].T, blk[:, TBLK:].T], axis=1)


def _tc_pair_transpose(emb_t):
    return pl.pallas_call(
        _transpose_body,
        grid=(NTBLK,),
        in_specs=[pl.BlockSpec((D, 2 * TBLK), lambda i: (0, i))],
        out_specs=pl.BlockSpec((TBLK, 2 * D), lambda i: (i, 0)),
        out_shape=jax.ShapeDtypeStruct((NTBLK * TBLK, 2 * D), jnp.float32),
        compiler_params=pltpu.CompilerParams(fuse_transposed_lhs_in_matmul=True),
        name="cbow_tc_pairT",
    )(emb_t)


def _head_body(pooled_ref, state_ref, stW_ref, stb_ref,
               W1_ref, b1_ref, W2_ref, b2_ref, out_ref):
    x = pooled_ref[:, :D] * (1.0 / H)
    x += lax.dot_general(state_ref[...], stW_ref[...],
                         (((1,), (1,)), ((), ())),
                         preferred_element_type=jnp.float32)
    x += stb_ref[...]
    h = jnp.maximum(x, 0.0)
    h = lax.dot_general(h, W1_ref[...], (((1,), (1,)), ((), ())),
                        preferred_element_type=jnp.float32) + b1_ref[...]
    h = jnp.maximum(h, 0.0)
    out_ref[...] = lax.dot_general(h, W2_ref[...], (((1,), (1,)), ((), ())),
                                   preferred_element_type=jnp.float32) + b2_ref[...]


def _tc_head(pooled, state, state_W, state_b, W1, b1, W2, b2):
    blk = 2048
    grid = (B // blk,)
    full = lambda shape: pl.BlockSpec(shape, lambda i: (0,) * len(shape))
    return pl.pallas_call(
        _head_body,
        grid=grid,
        in_specs=[
            pl.BlockSpec((blk, 2 * D), lambda i: (i, 0)),
            pl.BlockSpec((blk, 3), lambda i: (i, 0)),
            full((D, 3)),
            full((1, D)),
            full((D // 2, D)),
            full((1, D // 2)),
            full((3, D // 2)),
            full((1, 3)),
        ],
        out_specs=pl.BlockSpec((blk, 3), lambda i: (i, 0)),
        out_shape=jax.ShapeDtypeStruct((B, 3), jnp.float32),
        name="cbow_tc_head",
    )(pooled, state, state_W, state_b.reshape(1, D), W1,
      b1.reshape(1, D // 2), W2, b2.reshape(1, 3))


def kernel(players, state, emb_table, state_W, state_b, W1, b1, W2, b2):
    pflat = players.astype(jnp.int32).reshape(-1)
    pidx = ((pflat // (2 * TBLK)) * TBLK
            + pflat % TBLK).reshape(NW, IDX_ROWS, IDX_PER_GATHER)
    roff = (((pflat // TBLK) % 2) * D).reshape(NW, N_CHUNK, CHUNK * H)
    tbl2 = _tc_pair_transpose(emb_table.T)
    pooled = _sc_pool(pidx, roff, tbl2)
    return _tc_head(pooled, state, state_W, state_b, W1, b1, W2, b2)


# speedup vs baseline: 1.9546x; 1.1696x over previous
import jax
import jax.numpy as jnp
from jax import lax
from jax.experimental import pallas as pl
from jax.experimental.pallas import tpu as pltpu
from jax.experimental.pallas import tpu_sc as plsc

D = 64
B = 16384
H = 20
NC, NS, L = 2, 16, 16
NW = NC * NS
B_PER_W = B // NW
CHUNK = 16
N_CHUNK = B_PER_W // CHUNK
IDX_PER_GATHER = 64
G_PER_CHUNK = CHUNK * H // IDX_PER_GATHER
IDX_ROWS = B_PER_W * H // IDX_PER_GATHER
NUM_ROWS = 1000000
TBLK = 4096
NTBLK = -(-NUM_ROWS // (2 * TBLK))


def _pool_body(pidx_hbm, roff_hbm, tbl_hbm, out_hbm,
               idx_v, roff_v, buf, acc, gsem):
    wid = lax.axis_index("s") * NC + lax.axis_index("c")
    item_base = wid * B_PER_W

    pltpu.sync_copy(pidx_hbm.at[wid], idx_v)
    pltpu.sync_copy(roff_hbm.at[wid], roff_v)

    def fire(c, slot):
        for g in range(G_PER_CHUNK):
            pltpu.async_copy(
                tbl_hbm.at[idx_v.at[c * G_PER_CHUNK + g]],
                buf.at[slot, pl.ds(g * IDX_PER_GATHER, IDX_PER_GATHER)],
                gsem)

    def drain(slot):
        for g in range(G_PER_CHUNK):
            pltpu.make_async_copy(
                tbl_hbm.at[idx_v.at[g]],
                buf.at[slot, pl.ds(g * IDX_PER_GATHER, IDX_PER_GATHER)],
                gsem).wait()

    fire(0, 0)

    def chunk_body(c, _):
        slot = lax.rem(c, 2)

        @pl.when(c + 1 < N_CHUNK)
        def _():
            fire(c + 1, 1 - slot)

        drain(slot)
        ssplat = jnp.full((L,), slot, jnp.int32)
        csplat = jnp.full((L,), c, jnp.int32)

        def item_body(i, _):
            accs = [jnp.zeros((L,), jnp.float32) for _ in range(D // L)]
            for j in range(H):
                r = i * H + j
                rsplat = jnp.full((L,), r, jnp.int32)
                off = plsc.load_gather(roff_v, [csplat, rsplat])
                for k in range(D // L):
                    col = off + (k * L + lax.iota(jnp.int32, L))
                    accs[k] += plsc.load_gather(buf, [ssplat, rsplat, col])
            for k in range(D // L):
                acc[i, pl.ds(k * L, L)] = accs[k]
            return 0

        lax.fori_loop(0, CHUNK, item_body, 0)
        pltpu.sync_copy(acc, out_hbm.at[pl.ds(item_base + c * CHUNK, CHUNK)])
        return 0

    lax.fori_loop(0, N_CHUNK, chunk_body, 0)


def _sc_pool(pidx, roff, tbl2):
    mesh = plsc.VectorSubcoreMesh(core_axis_name="c", subcore_axis_name="s")
    return pl.kernel(
        _pool_body,
        out_type=jax.ShapeDtypeStruct((B, 2 * D), jnp.float32),
        mesh=mesh,
        scratch_types=[
            pltpu.VMEM((IDX_ROWS, IDX_PER_GATHER), jnp.int32),
            pltpu.VMEM((N_CHUNK, CHUNK * H), jnp.int32),
            pltpu.VMEM((2, CHUNK * H, 2 * D), jnp.float32),
            pltpu.VMEM((CHUNK, 2 * D), jnp.float32),
            pltpu.SemaphoreType.DMA,
        ],
        compiler_params=pltpu.CompilerParams(use_tc_tiling_on_sc=True,
                                             needs_layout_passes=False),
        name="cbow_sc_pool",
    )(pidx, roff, tbl2)


def _transpose_body(in_ref, out_ref):
    blk = in_ref[...]
    out_ref[...] = jnp.concatenate(
        [blk[:, :TBLK].T, blk[:, TBLK:].T], axis=1)


def _tc_pair_transpose(emb_t):
    return pl.pallas_call(
        _transpose_body,
        grid=(NTBLK,),
        in_specs=[pl.BlockSpec((D, 2 * TBLK), lambda i: (0, i))],
        out_specs=pl.BlockSpec((TBLK, 2 * D), lambda i: (i, 0)),
        out_shape=jax.ShapeDtypeStruct((NTBLK * TBLK, 2 * D), jnp.float32),
        compiler_params=pltpu.CompilerParams(fuse_transposed_lhs_in_matmul=True),
        name="cbow_tc_pairT",
    )(emb_t)


def _head_body(pooled_ref, state_ref, stW_ref, stb_ref,
               W1_ref, b1_ref, W2_ref, b2_ref, out_ref):
    x = pooled_ref[:, :D] * (1.0 / H)
    x += lax.dot_general(state_ref[...], stW_ref[...],
                         (((1,), (1,)), ((), ())),
                         preferred_element_type=jnp.float32)
    x += stb_ref[...]
    h = jnp.maximum(x, 0.0)
    h = lax.dot_general(h, W1_ref[...], (((1,), (1,)), ((), ())),
                        preferred_element_type=jnp.float32) + b1_ref[...]
    h = jnp.maximum(h, 0.0)
    out_ref[...] = lax.dot_general(h, W2_ref[...], (((1,), (1,)), ((), ())),
                                   preferred_element_type=jnp.float32) + b2_ref[...]


def _tc_head(pooled, state, state_W, state_b, W1, b1, W2, b2):
    blk = 2048
    grid = (B // blk,)
    full = lambda shape: pl.BlockSpec(shape, lambda i: (0,) * len(shape))
    return pl.pallas_call(
        _head_body,
        grid=grid,
        in_specs=[
            pl.BlockSpec((blk, 2 * D), lambda i: (i, 0)),
            pl.BlockSpec((blk, 3), lambda i: (i, 0)),
            full((D, 3)),
            full((1, D)),
            full((D // 2, D)),
            full((1, D // 2)),
            full((3, D // 2)),
            full((1, 3)),
        ],
        out_specs=pl.BlockSpec((blk, 3), lambda i: (i, 0)),
        out_shape=jax.ShapeDtypeStruct((B, 3), jnp.float32),
        name="cbow_tc_head",
    )(pooled, state, state_W, state_b.reshape(1, D), W1,
      b1.reshape(1, D // 2), W2, b2.reshape(1, 3))


def kernel(players, state, emb_table, state_W, state_b, W1, b1, W2, b2):
    pflat = players.astype(jnp.int32).reshape(-1)
    pidx = ((pflat // (2 * TBLK)) * TBLK
            + pflat % TBLK).reshape(NW, IDX_ROWS, IDX_PER_GATHER)
    roff = (((pflat // TBLK) % 2) * D).reshape(NW, N_CHUNK, CHUNK * H)
    tbl2 = _tc_pair_transpose(emb_table.T)
    pooled = _sc_pool(pidx, roff, tbl2)
    return _tc_head(pooled, state, state_W, state_b, W1, b1, W2, b2)


# speedup vs baseline: 2.1344x; 1.0920x over previous
import jax
import jax.numpy as jnp
from jax import lax
from jax.experimental import pallas as pl
from jax.experimental.pallas import tpu as pltpu
from jax.experimental.pallas import tpu_sc as plsc

D = 64
B = 16384
H = 20
NC, NS, L = 2, 16, 16
NW = NC * NS
B_PER_W = B // NW
CHUNK = 16
N_CHUNK = B_PER_W // CHUNK
IDX_PER_GATHER = 64
G_PER_CHUNK = CHUNK * H // IDX_PER_GATHER
IDX_ROWS = B_PER_W * H // IDX_PER_GATHER
NUM_ROWS = 1000000
TBLK = 8192
NTBLK = -(-NUM_ROWS // (2 * TBLK))


def _pool_body(pidx_hbm, roff_hbm, tbl_hbm, out_hbm,
               idx_v, roff_v, buf, acc, gsem):
    wid = lax.axis_index("s") * NC + lax.axis_index("c")
    item_base = wid * B_PER_W

    pltpu.sync_copy(pidx_hbm.at[wid], idx_v)
    pltpu.sync_copy(roff_hbm.at[wid], roff_v)

    def fire(c, slot):
        for g in range(G_PER_CHUNK):
            pltpu.async_copy(
                tbl_hbm.at[idx_v.at[c * G_PER_CHUNK + g]],
                buf.at[slot, pl.ds(g * IDX_PER_GATHER, IDX_PER_GATHER)],
                gsem)

    def drain(slot):
        for g in range(G_PER_CHUNK):
            pltpu.make_async_copy(
                tbl_hbm.at[idx_v.at[g]],
                buf.at[slot, pl.ds(g * IDX_PER_GATHER, IDX_PER_GATHER)],
                gsem).wait()

    fire(0, 0)

    def chunk_body(c, _):
        slot = lax.rem(c, 2)

        @pl.when(c + 1 < N_CHUNK)
        def _():
            fire(c + 1, 1 - slot)

        drain(slot)
        ssplat = jnp.full((L,), slot, jnp.int32)
        csplat = jnp.full((L,), c, jnp.int32)

        def item_body(i, _):
            accs = [jnp.zeros((L,), jnp.float32) for _ in range(D // L)]
            for j in range(H):
                r = i * H + j
                rsplat = jnp.full((L,), r, jnp.int32)
                off = plsc.load_gather(roff_v, [csplat, rsplat])
                for k in range(D // L):
                    col = off + (k * L + lax.iota(jnp.int32, L))
                    accs[k] += plsc.load_gather(buf, [ssplat, rsplat, col])
            for k in range(D // L):
                acc[i, pl.ds(k * L, L)] = accs[k]
            return 0

        lax.fori_loop(0, CHUNK, item_body, 0)
        pltpu.sync_copy(acc, out_hbm.at[pl.ds(item_base + c * CHUNK, CHUNK)])
        return 0

    lax.fori_loop(0, N_CHUNK, chunk_body, 0)


def _sc_pool(pidx, roff, tbl2):
    mesh = plsc.VectorSubcoreMesh(core_axis_name="c", subcore_axis_name="s")
    return pl.kernel(
        _pool_body,
        out_type=jax.ShapeDtypeStruct((B, 2 * D), jnp.float32),
        mesh=mesh,
        scratch_types=[
            pltpu.VMEM((IDX_ROWS, IDX_PER_GATHER), jnp.int32),
            pltpu.VMEM((N_CHUNK, CHUNK * H), jnp.int32),
            pltpu.VMEM((2, CHUNK * H, 2 * D), jnp.float32),
            pltpu.VMEM((CHUNK, 2 * D), jnp.float32),
            pltpu.SemaphoreType.DMA,
        ],
        compiler_params=pltpu.CompilerParams(use_tc_tiling_on_sc=True,
                                             needs_layout_passes=False),
        name="cbow_sc_pool",
    )(pidx, roff, tbl2)


def _transpose_body(in_ref, out_ref):
    blk = in_ref[...]
    out_ref[...] = jnp.concatenate(
        [blk[:, :TBLK].T, blk[:, TBLK:].T], axis=1)


def _tc_pair_transpose(emb_t):
    return pl.pallas_call(
        _transpose_body,
        grid=(NTBLK,),
        in_specs=[pl.BlockSpec((D, 2 * TBLK), lambda i: (0, i))],
        out_specs=pl.BlockSpec((TBLK, 2 * D), lambda i: (i, 0)),
        out_shape=jax.ShapeDtypeStruct((NTBLK * TBLK, 2 * D), jnp.float32),
        compiler_params=pltpu.CompilerParams(fuse_transposed_lhs_in_matmul=True),
        name="cbow_tc_pairT",
    )(emb_t)


def _head_body(pooled_ref, state_ref, stW_ref, stb_ref,
               W1_ref, b1_ref, W2_ref, b2_ref, out_ref):
    x = pooled_ref[:, :D] * (1.0 / H)
    x += lax.dot_general(state_ref[...], stW_ref[...],
                         (((1,), (1,)), ((), ())),
                         preferred_element_type=jnp.float32)
    x += stb_ref[...]
    h = jnp.maximum(x, 0.0)
    h = lax.dot_general(h, W1_ref[...], (((1,), (1,)), ((), ())),
                        preferred_element_type=jnp.float32) + b1_ref[...]
    h = jnp.maximum(h, 0.0)
    out_ref[...] = lax.dot_general(h, W2_ref[...], (((1,), (1,)), ((), ())),
                                   preferred_element_type=jnp.float32) + b2_ref[...]


def _tc_head(pooled, state, state_W, state_b, W1, b1, W2, b2):
    blk = 2048
    grid = (B // blk,)
    full = lambda shape: pl.BlockSpec(shape, lambda i: (0,) * len(shape))
    return pl.pallas_call(
        _head_body,
        grid=grid,
        in_specs=[
            pl.BlockSpec((blk, 2 * D), lambda i: (i, 0)),
            pl.BlockSpec((blk, 3), lambda i: (i, 0)),
            full((D, 3)),
            full((1, D)),
            full((D // 2, D)),
            full((1, D // 2)),
            full((3, D // 2)),
            full((1, 3)),
        ],
        out_specs=pl.BlockSpec((blk, 3), lambda i: (i, 0)),
        out_shape=jax.ShapeDtypeStruct((B, 3), jnp.float32),
        name="cbow_tc_head",
    )(pooled, state, state_W, state_b.reshape(1, D), W1,
      b1.reshape(1, D // 2), W2, b2.reshape(1, 3))


def kernel(players, state, emb_table, state_W, state_b, W1, b1, W2, b2):
    pflat = players.astype(jnp.int32).reshape(-1)
    pidx = ((pflat // (2 * TBLK)) * TBLK
            + pflat % TBLK).reshape(NW, IDX_ROWS, IDX_PER_GATHER)
    roff = (((pflat // TBLK) % 2) * D).reshape(NW, N_CHUNK, CHUNK * H)
    tbl2 = _tc_pair_transpose(emb_table.T)
    pooled = _sc_pool(pidx, roff, tbl2)
    return _tc_head(pooled, state, state_W, state_b, W1, b1, W2, b2)


# speedup vs baseline: 2.2219x; 1.0410x over previous
import jax
import jax.numpy as jnp
from jax import lax
from jax.experimental import pallas as pl
from jax.experimental.pallas import tpu as pltpu
from jax.experimental.pallas import tpu_sc as plsc

D = 64
B = 16384
H = 20
NC, NS, L = 2, 16, 16
NW = NC * NS
B_PER_W = B // NW
CHUNK = 16
N_CHUNK = B_PER_W // CHUNK
IDX_PER_GATHER = 64
G_PER_CHUNK = CHUNK * H // IDX_PER_GATHER
IDX_ROWS = B_PER_W * H // IDX_PER_GATHER
NUM_ROWS = 1000000
TBLK = 16384
NTBLK = -(-NUM_ROWS // (2 * TBLK))


def _pool_body(pidx_hbm, roff_hbm, tbl_hbm, out_hbm,
               idx_v, roff_v, buf, acc, gsem):
    wid = lax.axis_index("s") * NC + lax.axis_index("c")
    item_base = wid * B_PER_W

    pltpu.sync_copy(pidx_hbm.at[wid], idx_v)
    pltpu.sync_copy(roff_hbm.at[wid], roff_v)

    def fire(c, slot):
        for g in range(G_PER_CHUNK):
            pltpu.async_copy(
                tbl_hbm.at[idx_v.at[c * G_PER_CHUNK + g]],
                buf.at[slot, pl.ds(g * IDX_PER_GATHER, IDX_PER_GATHER)],
                gsem)

    def drain(slot):
        for g in range(G_PER_CHUNK):
            pltpu.make_async_copy(
                tbl_hbm.at[idx_v.at[g]],
                buf.at[slot, pl.ds(g * IDX_PER_GATHER, IDX_PER_GATHER)],
                gsem).wait()

    fire(0, 0)

    def chunk_body(c, _):
        slot = lax.rem(c, 2)

        @pl.when(c + 1 < N_CHUNK)
        def _():
            fire(c + 1, 1 - slot)

        drain(slot)
        ssplat = jnp.full((L,), slot, jnp.int32)
        csplat = jnp.full((L,), c, jnp.int32)

        def item_body(i, _):
            accs = [jnp.zeros((L,), jnp.float32) for _ in range(D // L)]
            for j in range(H):
                r = i * H + j
                rsplat = jnp.full((L,), r, jnp.int32)
                off = plsc.load_gather(roff_v, [csplat, rsplat])
                for k in range(D // L):
                    col = off + (k * L + lax.iota(jnp.int32, L))
                    accs[k] += plsc.load_gather(buf, [ssplat, rsplat, col])
            for k in range(D // L):
                acc[i, pl.ds(k * L, L)] = accs[k]
            return 0

        lax.fori_loop(0, CHUNK, item_body, 0)
        pltpu.sync_copy(acc, out_hbm.at[pl.ds(item_base + c * CHUNK, CHUNK)])
        return 0

    lax.fori_loop(0, N_CHUNK, chunk_body, 0)


def _sc_pool(pidx, roff, tbl2):
    mesh = plsc.VectorSubcoreMesh(core_axis_name="c", subcore_axis_name="s")
    return pl.kernel(
        _pool_body,
        out_type=jax.ShapeDtypeStruct((B, 2 * D), jnp.float32),
        mesh=mesh,
        scratch_types=[
            pltpu.VMEM((IDX_ROWS, IDX_PER_GATHER), jnp.int32),
            pltpu.VMEM((N_CHUNK, CHUNK * H), jnp.int32),
            pltpu.VMEM((2, CHUNK * H, 2 * D), jnp.float32),
            pltpu.VMEM((CHUNK, 2 * D), jnp.float32),
            pltpu.SemaphoreType.DMA,
        ],
        compiler_params=pltpu.CompilerParams(use_tc_tiling_on_sc=True,
                                             needs_layout_passes=False),
        name="cbow_sc_pool",
    )(pidx, roff, tbl2)


def _transpose_body(in_ref, out_ref):
    blk = in_ref[...]
    out_ref[...] = jnp.concatenate(
        [blk[:, :TBLK].T, blk[:, TBLK:].T], axis=1)


def _tc_pair_transpose(emb_t):
    return pl.pallas_call(
        _transpose_body,
        grid=(NTBLK,),
        in_specs=[pl.BlockSpec((D, 2 * TBLK), lambda i: (0, i))],
        out_specs=pl.BlockSpec((TBLK, 2 * D), lambda i: (i, 0)),
        out_shape=jax.ShapeDtypeStruct((NTBLK * TBLK, 2 * D), jnp.float32),
        compiler_params=pltpu.CompilerParams(fuse_transposed_lhs_in_matmul=True),
        name="cbow_tc_pairT",
    )(emb_t)


def _head_body(pooled_ref, state_ref, stW_ref, stb_ref,
               W1_ref, b1_ref, W2_ref, b2_ref, out_ref):
    x = pooled_ref[:, :D] * (1.0 / H)
    x += lax.dot_general(state_ref[...], stW_ref[...],
                         (((1,), (1,)), ((), ())),
                         preferred_element_type=jnp.float32)
    x += stb_ref[...]
    h = jnp.maximum(x, 0.0)
    h = lax.dot_general(h, W1_ref[...], (((1,), (1,)), ((), ())),
                        preferred_element_type=jnp.float32) + b1_ref[...]
    h = jnp.maximum(h, 0.0)
    out_ref[...] = lax.dot_general(h, W2_ref[...], (((1,), (1,)), ((), ())),
                                   preferred_element_type=jnp.float32) + b2_ref[...]


def _tc_head(pooled, state, state_W, state_b, W1, b1, W2, b2):
    blk = 2048
    grid = (B // blk,)
    full = lambda shape: pl.BlockSpec(shape, lambda i: (0,) * len(shape))
    return pl.pallas_call(
        _head_body,
        grid=grid,
        in_specs=[
            pl.BlockSpec((blk, 2 * D), lambda i: (i, 0)),
            pl.BlockSpec((blk, 3), lambda i: (i, 0)),
            full((D, 3)),
            full((1, D)),
            full((D // 2, D)),
            full((1, D // 2)),
            full((3, D // 2)),
            full((1, 3)),
        ],
        out_specs=pl.BlockSpec((blk, 3), lambda i: (i, 0)),
        out_shape=jax.ShapeDtypeStruct((B, 3), jnp.float32),
        name="cbow_tc_head",
    )(pooled, state, state_W, state_b.reshape(1, D), W1,
      b1.reshape(1, D // 2), W2, b2.reshape(1, 3))


def kernel(players, state, emb_table, state_W, state_b, W1, b1, W2, b2):
    pflat = players.astype(jnp.int32).reshape(-1)
    pidx = ((pflat // (2 * TBLK)) * TBLK
            + pflat % TBLK).reshape(NW, IDX_ROWS, IDX_PER_GATHER)
    roff = (((pflat // TBLK) % 2) * D).reshape(NW, N_CHUNK, CHUNK * H)
    tbl2 = _tc_pair_transpose(emb_table.T)
    pooled = _sc_pool(pidx, roff, tbl2)
    return _tc_head(pooled, state, state_W, state_b, W1, b1, W2, b2)


# speedup vs baseline: 2.2294x; 1.0034x over previous
import jax
import jax.numpy as jnp
from jax import lax
from jax.experimental import pallas as pl
from jax.experimental.pallas import tpu as pltpu
from jax.experimental.pallas import tpu_sc as plsc

D = 64
B = 16384
H = 20
NC, NS, L = 2, 16, 16
NW = NC * NS
B_PER_W = B // NW
CHUNK = 16
N_CHUNK = B_PER_W // CHUNK
IDX_PER_GATHER = 64
G_PER_CHUNK = CHUNK * H // IDX_PER_GATHER
IDX_ROWS = B_PER_W * H // IDX_PER_GATHER
NUM_ROWS = 1000000
TBLK = 16384
NTBLK = -(-NUM_ROWS // (2 * TBLK))


def _pool_body(pidx_hbm, roff_hbm, tbl_hbm, out_hbm,
               idx_v, roff_v, buf, acc, gsem):
    wid = lax.axis_index("s") * NC + lax.axis_index("c")
    item_base = wid * B_PER_W

    pltpu.sync_copy(pidx_hbm.at[wid], idx_v)
    pltpu.sync_copy(roff_hbm.at[wid], roff_v)

    def fire(c, slot):
        for g in range(G_PER_CHUNK):
            pltpu.async_copy(
                tbl_hbm.at[idx_v.at[c * G_PER_CHUNK + g]],
                buf.at[slot, pl.ds(g * IDX_PER_GATHER, IDX_PER_GATHER)],
                gsem)

    def drain(slot):
        for g in range(G_PER_CHUNK):
            pltpu.make_async_copy(
                tbl_hbm.at[idx_v.at[g]],
                buf.at[slot, pl.ds(g * IDX_PER_GATHER, IDX_PER_GATHER)],
                gsem).wait()

    fire(0, 0)

    def chunk_body(c, _):
        slot = lax.rem(c, 2)

        @pl.when(c + 1 < N_CHUNK)
        def _():
            fire(c + 1, 1 - slot)

        drain(slot)
        ssplat = jnp.full((L,), slot, jnp.int32)
        csplat = jnp.full((L,), c, jnp.int32)

        def item_body(i2, _):
            for u in range(2):
                i = 2 * i2 + u
                accs = [jnp.zeros((L,), jnp.float32) for _ in range(D // L)]
                accs2 = [jnp.zeros((L,), jnp.float32) for _ in range(D // L)]
                for j in range(H):
                    r = i * H + j
                    rsplat = jnp.full((L,), r, jnp.int32)
                    off = plsc.load_gather(roff_v, [csplat, rsplat])
                    tgt = accs if j % 2 == 0 else accs2
                    for k in range(D // L):
                        col = off + (k * L + lax.iota(jnp.int32, L))
                        tgt[k] += plsc.load_gather(buf, [ssplat, rsplat, col])
                for k in range(D // L):
                    acc[i, pl.ds(k * L, L)] = accs[k] + accs2[k]
            return 0

        lax.fori_loop(0, CHUNK // 2, item_body, 0)
        pltpu.sync_copy(acc, out_hbm.at[pl.ds(item_base + c * CHUNK, CHUNK)])
        return 0

    lax.fori_loop(0, N_CHUNK, chunk_body, 0)


def _sc_pool(pidx, roff, tbl2):
    mesh = plsc.VectorSubcoreMesh(core_axis_name="c", subcore_axis_name="s")
    return pl.kernel(
        _pool_body,
        out_type=jax.ShapeDtypeStruct((B, 2 * D), jnp.float32),
        mesh=mesh,
        scratch_types=[
            pltpu.VMEM((IDX_ROWS, IDX_PER_GATHER), jnp.int32),
            pltpu.VMEM((N_CHUNK, CHUNK * H), jnp.int32),
            pltpu.VMEM((2, CHUNK * H, 2 * D), jnp.float32),
            pltpu.VMEM((CHUNK, 2 * D), jnp.float32),
            pltpu.SemaphoreType.DMA,
        ],
        compiler_params=pltpu.CompilerParams(use_tc_tiling_on_sc=True,
                                             needs_layout_passes=False),
        name="cbow_sc_pool",
    )(pidx, roff, tbl2)


def _transpose_body(in_ref, out_ref):
    blk = in_ref[...]
    out_ref[...] = jnp.concatenate(
        [blk[:, :TBLK].T, blk[:, TBLK:].T], axis=1)


def _tc_pair_transpose(emb_t):
    return pl.pallas_call(
        _transpose_body,
        grid=(NTBLK,),
        in_specs=[pl.BlockSpec((D, 2 * TBLK), lambda i: (0, i))],
        out_specs=pl.BlockSpec((TBLK, 2 * D), lambda i: (i, 0)),
        out_shape=jax.ShapeDtypeStruct((NTBLK * TBLK, 2 * D), jnp.float32),
        compiler_params=pltpu.CompilerParams(fuse_transposed_lhs_in_matmul=True),
        name="cbow_tc_pairT",
    )(emb_t)


def _head_body(pooled_ref, state_ref, stW_ref, stb_ref,
               W1_ref, b1_ref, W2_ref, b2_ref, out_ref):
    x = pooled_ref[:, :D] * (1.0 / H)
    x += lax.dot_general(state_ref[...], stW_ref[...],
                         (((1,), (1,)), ((), ())),
                         preferred_element_type=jnp.float32)
    x += stb_ref[...]
    h = jnp.maximum(x, 0.0)
    h = lax.dot_general(h, W1_ref[...], (((1,), (1,)), ((), ())),
                        preferred_element_type=jnp.float32) + b1_ref[...]
    h = jnp.maximum(h, 0.0)
    out_ref[...] = lax.dot_general(h, W2_ref[...], (((1,), (1,)), ((), ())),
                                   preferred_element_type=jnp.float32) + b2_ref[...]


def _tc_head(pooled, state, state_W, state_b, W1, b1, W2, b2):
    blk = 2048
    grid = (B // blk,)
    full = lambda shape: pl.BlockSpec(shape, lambda i: (0,) * len(shape))
    return pl.pallas_call(
        _head_body,
        grid=grid,
        in_specs=[
            pl.BlockSpec((blk, 2 * D), lambda i: (i, 0)),
            pl.BlockSpec((blk, 3), lambda i: (i, 0)),
            full((D, 3)),
            full((1, D)),
            full((D // 2, D)),
            full((1, D // 2)),
            full((3, D // 2)),
            full((1, 3)),
        ],
        out_specs=pl.BlockSpec((blk, 3), lambda i: (i, 0)),
        out_shape=jax.ShapeDtypeStruct((B, 3), jnp.float32),
        name="cbow_tc_head",
    )(pooled, state, state_W, state_b.reshape(1, D), W1,
      b1.reshape(1, D // 2), W2, b2.reshape(1, 3))


def kernel(players, state, emb_table, state_W, state_b, W1, b1, W2, b2):
    pflat = players.astype(jnp.int32).reshape(-1)
    pidx = ((pflat // (2 * TBLK)) * TBLK
            + pflat % TBLK).reshape(NW, IDX_ROWS, IDX_PER_GATHER)
    roff = (((pflat // TBLK) % 2) * D).reshape(NW, N_CHUNK, CHUNK * H)
    tbl2 = _tc_pair_transpose(emb_table.T)
    pooled = _sc_pool(pidx, roff, tbl2)
    return _tc_head(pooled, state, state_W, state_b, W1, b1, W2, b2)


# speedup vs baseline: 2.3157x; 1.0387x over previous
import jax
import jax.numpy as jnp
from jax import lax
from jax.experimental import pallas as pl
from jax.experimental.pallas import tpu as pltpu
from jax.experimental.pallas import tpu_sc as plsc

D = 64
B = 16384
H = 20
NC, NS, L = 2, 16, 16
NW = NC * NS
B_PER_W = B // NW
CHUNK = 16
N_CHUNK = B_PER_W // CHUNK
IDX_PER_GATHER = 64
G_PER_CHUNK = CHUNK * H // IDX_PER_GATHER
IDX_ROWS = B_PER_W * H // IDX_PER_GATHER
NUM_ROWS = 1000000
TBLK = 16384
NTBLK = -(-NUM_ROWS // (2 * TBLK))


def _pool_body(players_hbm, tbl_hbm, out_hbm,
               praw_v, idx_v, roff_v, buf, acc, gsem):
    wid = lax.axis_index("s") * NC + lax.axis_index("c")
    item_base = wid * B_PER_W

    pltpu.sync_copy(players_hbm.at[wid], praw_v)

    def build_body(v, _):
        p = praw_v[pl.ds(v * L, L)]
        pidx16 = lax.shift_left(lax.shift_right_logical(p, 15), 14) + \
            lax.bitwise_and(p, TBLK - 1)
        roff16 = lax.shift_left(
            lax.bitwise_and(lax.shift_right_logical(p, 14), 1), 6)
        idx_v[lax.shift_right_logical(v, 2),
              pl.ds(lax.bitwise_and(v, 3) * L, L)] = pidx16
        roff_v[pl.ds(v * L, L)] = roff16
        return 0

    lax.fori_loop(0, B_PER_W * H // L, build_body, 0)

    def fire(c, slot):
        for g in range(G_PER_CHUNK):
            pltpu.async_copy(
                tbl_hbm.at[idx_v.at[c * G_PER_CHUNK + g]],
                buf.at[slot, pl.ds(g * IDX_PER_GATHER, IDX_PER_GATHER)],
                gsem)

    def drain(slot):
        for g in range(G_PER_CHUNK):
            pltpu.make_async_copy(
                tbl_hbm.at[idx_v.at[g]],
                buf.at[slot, pl.ds(g * IDX_PER_GATHER, IDX_PER_GATHER)],
                gsem).wait()

    fire(0, 0)

    def chunk_body(c, _):
        slot = lax.rem(c, 2)

        @pl.when(c + 1 < N_CHUNK)
        def _():
            fire(c + 1, 1 - slot)

        drain(slot)
        ssplat = jnp.full((L,), slot, jnp.int32)
        cbase = c * CHUNK * H

        def item_body(i2, _):
            for u in range(2):
                i = 2 * i2 + u
                accs = [jnp.zeros((L,), jnp.float32) for _ in range(D // L)]
                accs2 = [jnp.zeros((L,), jnp.float32) for _ in range(D // L)]
                for j in range(H):
                    r = i * H + j
                    rsplat = jnp.full((L,), r, jnp.int32)
                    off = plsc.load_gather(roff_v, [jnp.full((L,), cbase + r,
                                                            jnp.int32)])
                    tgt = accs if j % 2 == 0 else accs2
                    for k in range(D // L):
                        col = off + (k * L + lax.iota(jnp.int32, L))
                        tgt[k] += plsc.load_gather(buf, [ssplat, rsplat, col])
                for k in range(D // L):
                    acc[i, pl.ds(k * L, L)] = accs[k] + accs2[k]
            return 0

        lax.fori_loop(0, CHUNK // 2, item_body, 0)
        pltpu.sync_copy(acc, out_hbm.at[pl.ds(item_base + c * CHUNK, CHUNK)])
        return 0

    lax.fori_loop(0, N_CHUNK, chunk_body, 0)


def _sc_pool(players_w, tbl2):
    mesh = plsc.VectorSubcoreMesh(core_axis_name="c", subcore_axis_name="s")
    return pl.kernel(
        _pool_body,
        out_type=jax.ShapeDtypeStruct((B, 2 * D), jnp.float32),
        mesh=mesh,
        scratch_types=[
            pltpu.VMEM((B_PER_W * H,), jnp.int32),
            pltpu.VMEM((IDX_ROWS, IDX_PER_GATHER), jnp.int32),
            pltpu.VMEM((B_PER_W * H,), jnp.int32),
            pltpu.VMEM((2, CHUNK * H, 2 * D), jnp.float32),
            pltpu.VMEM((CHUNK, 2 * D), jnp.float32),
            pltpu.SemaphoreType.DMA,
        ],
        compiler_params=pltpu.CompilerParams(use_tc_tiling_on_sc=True,
                                             needs_layout_passes=False),
        name="cbow_sc_pool",
    )(players_w, tbl2)


def _transpose_body(in_ref, out_ref):
    blk = in_ref[...]
    out_ref[...] = jnp.concatenate(
        [blk[:, :TBLK].T, blk[:, TBLK:].T], axis=1)


def _tc_pair_transpose(emb_t):
    return pl.pallas_call(
        _transpose_body,
        grid=(NTBLK,),
        in_specs=[pl.BlockSpec((D, 2 * TBLK), lambda i: (0, i))],
        out_specs=pl.BlockSpec((TBLK, 2 * D), lambda i: (i, 0)),
        out_shape=jax.ShapeDtypeStruct((NTBLK * TBLK, 2 * D), jnp.float32),
        compiler_params=pltpu.CompilerParams(fuse_transposed_lhs_in_matmul=True),
        name="cbow_tc_pairT",
    )(emb_t)


def _head_body(pooled_ref, state_ref, stW_ref, stb_ref,
               W1_ref, b1_ref, W2_ref, b2_ref, out_ref):
    x = pooled_ref[:, :D] * (1.0 / H)
    x += lax.dot_general(state_ref[...], stW_ref[...],
                         (((1,), (1,)), ((), ())),
                         preferred_element_type=jnp.float32)
    x += stb_ref[...]
    h = jnp.maximum(x, 0.0)
    h = lax.dot_general(h, W1_ref[...], (((1,), (1,)), ((), ())),
                        preferred_element_type=jnp.float32) + b1_ref[...]
    h = jnp.maximum(h, 0.0)
    out_ref[...] = lax.dot_general(h, W2_ref[...], (((1,), (1,)), ((), ())),
                                   preferred_element_type=jnp.float32) + b2_ref[...]


def _tc_head(pooled, state, state_W, state_b, W1, b1, W2, b2):
    blk = 2048
    grid = (B // blk,)
    full = lambda shape: pl.BlockSpec(shape, lambda i: (0,) * len(shape))
    return pl.pallas_call(
        _head_body,
        grid=grid,
        in_specs=[
            pl.BlockSpec((blk, 2 * D), lambda i: (i, 0)),
            pl.BlockSpec((blk, 3), lambda i: (i, 0)),
            full((D, 3)),
            full((1, D)),
            full((D // 2, D)),
            full((1, D // 2)),
            full((3, D // 2)),
            full((1, 3)),
        ],
        out_specs=pl.BlockSpec((blk, 3), lambda i: (i, 0)),
        out_shape=jax.ShapeDtypeStruct((B, 3), jnp.float32),
        name="cbow_tc_head",
    )(pooled, state, state_W, state_b.reshape(1, D), W1,
      b1.reshape(1, D // 2), W2, b2.reshape(1, 3))


def kernel(players, state, emb_table, state_W, state_b, W1, b1, W2, b2):
    players_w = players.astype(jnp.int32).reshape(NW, B_PER_W * H)
    tbl2 = _tc_pair_transpose(emb_table.T)
    pooled = _sc_pool(players_w, tbl2)
    return _tc_head(pooled, state, state_W, state_b, W1, b1, W2, b2)
